# baseline (speedup 1.0000x reference)
import functools
import jax
import jax.numpy as jnp
from jax import lax
from jax.experimental import pallas as pl
from jax.experimental.pallas import tpu as pltpu
from jax.experimental.pallas import tpu_sc as plsc

N = 10000
E = 320000
FT = 128
H = 8
DH = FT // H
HP = 16

NC = 2
NS = 16
NW = NC * NS
EP = E // NW
C = 80
NCH = EP // C
SR = 640
SRL = N - (NS - 1) * SR

_mesh = plsc.VectorSubcoreMesh(core_axis_name="c", subcore_axis_name="s")


def _split_rows(s, do):
    @pl.when(s < NS - 1)
    def _():
        do(s * SR, SR)

    @pl.when(s == NS - 1)
    def _():
        do((NS - 1) * SR, SRL)


def _node_prep_body(nft_ref, w1t_ref, w3t_ref, wa_ref, a2s_ref, a2d_ref,
                    psrc_ref, pdst_ref, nsw_ref, ndw_ref):
    x = nft_ref[...]
    psrc = jnp.dot(x, w1t_ref[...], preferred_element_type=jnp.float32)
    pdst = jnp.dot(x, w3t_ref[...], preferred_element_type=jnp.float32)
    psrc_ref[...] = psrc
    pdst_ref[...] = pdst
    nsw_ref[...] = (jnp.dot(x, wa_ref[...], preferred_element_type=jnp.float32)
                    + jnp.dot(psrc, a2s_ref[...], preferred_element_type=jnp.float32))
    ndw_ref[...] = jnp.dot(pdst, a2d_ref[...], preferred_element_type=jnp.float32)


def _edge_prep_body(eft_ref, w2t_ref, b_ref, a2_ref, pe_ref, e2_ref):
    pe = jnp.dot(eft_ref[...], w2t_ref[...],
                 preferred_element_type=jnp.float32) + b_ref[...]
    pe_ref[...] = pe
    e2_ref[...] = jnp.dot(pe, a2_ref[...], preferred_element_type=jnp.float32)


def _final_body(accp_ref, dpart_ref, r_ref, pdst_ref, nft_ref, out_ref):
    d = jnp.dot(dpart_ref[0] + dpart_ref[1], r_ref[...],
                preferred_element_type=jnp.float32)
    inv = jnp.where(d > 0.0, 1.0 / d, 0.0)
    pd = jnp.where(d > 0.0, pdst_ref[...], 0.0)
    out_ref[...] = jnp.maximum(
        (accp_ref[0] + accp_ref[1]) * inv + pd + nft_ref[...], 0.0)


def _sc_edges(src_h, dst_h, psrc_h, nsw_h, ndw_h, pe_h, e2_h,
              p_h, accp_h,
              idx_s, idx_d, gs, gn, gd, pev, e2v, wbuf, acc, sem):
    c = lax.axis_index("c")
    s = lax.axis_index("s")
    wid = s * NC + c
    base_w = wid * EP

    def zrow(i, carry):
        def zcol(h, hc):
            wbuf[i, pl.ds(h * HP, HP)] = jnp.zeros((HP,), jnp.float32)
            return hc
        lax.fori_loop(0, FT // HP, zcol, 0)
        return carry
    lax.fori_loop(0, C, zrow, 0)

    def stage(off, size):
        def zcopy(j, carry):
            pltpu.sync_copy(wbuf, acc.at[pl.ds(off + j * C, C), :])
            return carry
        lax.fori_loop(0, size // C, zcopy, 0)
    _split_rows(s, stage)
    plsc.subcore_barrier()

    def chunk(k, carry):
        off = base_w + k * C
        pltpu.sync_copy(src_h.at[pl.ds(off, C)], idx_s)
        pltpu.sync_copy(dst_h.at[pl.ds(off, C)], idx_d)
        cp1 = pltpu.async_copy(psrc_h.at[idx_s], gs, sem)
        cp2 = pltpu.async_copy(nsw_h.at[idx_s], gn, sem)
        cp3 = pltpu.async_copy(ndw_h.at[idx_d], gd, sem)
        cp4 = pltpu.async_copy(pe_h.at[pl.ds(off, C), :], pev, sem)
        cp5 = pltpu.async_copy(e2_h.at[pl.ds(off, C), :], e2v, sem)
        cp1.wait()
        cp2.wait()
        cp3.wait()
        cp4.wait()
        cp5.wait()

        def erow(i, icarry):
            a = gn[i, :] + gd[i, :] + e2v[i, :]
            a = jnp.where(a >= 0.0, a, a * 0.01)
            p = jnp.exp(a)
            e2v[i, :] = p
            for h in range(H):
                w = (gs[i, pl.ds(h * DH, DH)] + pev[i, pl.ds(h * DH, DH)]) * p[h]
                wbuf[i, pl.ds(h * DH, DH)] = w
            return icarry
        lax.fori_loop(0, C, erow, 0)

        pltpu.sync_copy(e2v, p_h.at[pl.ds(off, C), :])
        pltpu.sync_copy(wbuf, acc.at[idx_d], add=True)
        return carry
    lax.fori_loop(0, NCH, chunk, 0)

    plsc.subcore_barrier()

    def writeback(off, size):
        pltpu.sync_copy(acc.at[pl.ds(off, size), :],
                        accp_h.at[c, pl.ds(off, size), :])
    _split_rows(s, writeback)


def _sc_denom(dst_h, p_h,
              dpart_h,
              idx_d, pv, dsh, sem):
    c = lax.axis_index("c")
    s = lax.axis_index("s")
    wid = s * NC + c
    base_w = wid * EP

    def zrow(i, carry):
        pv[i, :] = jnp.zeros((HP,), jnp.float32)
        return carry
    lax.fori_loop(0, C, zrow, 0)

    def stage(off, size):
        def zcopy(j, carry):
            pltpu.sync_copy(pv, dsh.at[pl.ds(off + j * C, C), :])
            return carry
        lax.fori_loop(0, size // C, zcopy, 0)
    _split_rows(s, stage)
    plsc.subcore_barrier()

    def chunk(k, carry):
        off = base_w + k * C
        pltpu.sync_copy(dst_h.at[pl.ds(off, C)], idx_d)
        pltpu.sync_copy(p_h.at[pl.ds(off, C), :], pv)
        pltpu.sync_copy(pv, dsh.at[idx_d], add=True)
        return carry
    lax.fori_loop(0, NCH, chunk, 0)

    plsc.subcore_barrier()

    @pl.when(s == 0)
    def _():
        pltpu.sync_copy(dsh, dpart_h.at[c])


@jax.jit
def kernel(nft, eft, edge_index, W_pe, b_pe, W_a1, attn2):
    f32 = jnp.float32
    src = edge_index[0]
    dst = edge_index[1]
    W1T = W_pe[:, :FT].T
    W2T = W_pe[:, FT:2 * FT].T
    W3T = W_pe[:, 2 * FT:].T
    a2sq = attn2[0]
    A2 = (a2sq[:, :, None] * jnp.eye(H, dtype=f32)[:, None, :]).reshape(FT, H)
    zpad = jnp.zeros((FT, FT - H), f32)
    A2w = jnp.concatenate([A2, zpad], axis=1)
    Wa1w = jnp.concatenate([W_a1.T, zpad], axis=1)
    A2p = A2w[:, :HP]
    b2 = b_pe[None, :]

    bn = 2000
    psrc, pdst, nsw, ndw = pl.pallas_call(
        _node_prep_body,
        grid=(N // bn,),
        in_specs=[
            pl.BlockSpec((bn, FT), lambda i: (i, 0)),
            pl.BlockSpec((FT, FT), lambda i: (0, 0)),
            pl.BlockSpec((FT, FT), lambda i: (0, 0)),
            pl.BlockSpec((FT, FT), lambda i: (0, 0)),
            pl.BlockSpec((FT, FT), lambda i: (0, 0)),
            pl.BlockSpec((FT, FT), lambda i: (0, 0)),
        ],
        out_specs=[
            pl.BlockSpec((bn, FT), lambda i: (i, 0)),
            pl.BlockSpec((bn, FT), lambda i: (i, 0)),
            pl.BlockSpec((bn, FT), lambda i: (i, 0)),
            pl.BlockSpec((bn, FT), lambda i: (i, 0)),
        ],
        out_shape=[
            jax.ShapeDtypeStruct((N, FT), f32),
            jax.ShapeDtypeStruct((N, FT), f32),
            jax.ShapeDtypeStruct((N, FT), f32),
            jax.ShapeDtypeStruct((N, FT), f32),
        ],
    )(nft, W1T, W3T, Wa1w, A2w, A2w)

    be = 2000
    pe, e216 = pl.pallas_call(
        _edge_prep_body,
        grid=(E // be,),
        in_specs=[
            pl.BlockSpec((be, FT), lambda i: (i, 0)),
            pl.BlockSpec((FT, FT), lambda i: (0, 0)),
            pl.BlockSpec((1, FT), lambda i: (0, 0)),
            pl.BlockSpec((FT, HP), lambda i: (0, 0)),
        ],
        out_specs=[
            pl.BlockSpec((be, FT), lambda i: (i, 0)),
            pl.BlockSpec((be, HP), lambda i: (i, 0)),
        ],
        out_shape=[
            jax.ShapeDtypeStruct((E, FT), f32),
            jax.ShapeDtypeStruct((E, HP), f32),
        ],
    )(eft, W2T, b2, A2p)

    p_exp, accp = pl.kernel(
        _sc_edges,
        out_type=[
            jax.ShapeDtypeStruct((E, HP), f32),
            jax.ShapeDtypeStruct((NC, N, FT), f32),
        ],
        mesh=_mesh,
        compiler_params=pltpu.CompilerParams(use_tc_tiling_on_sc=False),
        scratch_types=[
            pltpu.VMEM((C,), jnp.int32),
            pltpu.VMEM((C,), jnp.int32),
            pltpu.VMEM((C, FT), f32),
            pltpu.VMEM((C, HP), f32),
            pltpu.VMEM((C, HP), f32),
            pltpu.VMEM((C, FT), f32),
            pltpu.VMEM((C, HP), f32),
            pltpu.VMEM((C, FT), f32),
            pltpu.VMEM_SHARED((N, FT), f32),
            pltpu.SemaphoreType.DMA,
        ],
    )(src, dst, psrc, nsw[:, :HP], ndw[:, :HP], pe, e216)

    dpart = pl.kernel(
        _sc_denom,
        out_type=jax.ShapeDtypeStruct((NC, N, HP), f32),
        mesh=_mesh,
        compiler_params=pltpu.CompilerParams(use_tc_tiling_on_sc=False),
        scratch_types=[
            pltpu.VMEM((C,), jnp.int32),
            pltpu.VMEM((C, HP), f32),
            pltpu.VMEM_SHARED((N, HP), f32),
            pltpu.SemaphoreType.DMA,
        ],
    )(dst, p_exp)

    R = jnp.concatenate(
        [jnp.kron(jnp.eye(H, dtype=f32), jnp.ones((1, DH), f32)),
         jnp.zeros((HP - H, FT), f32)], axis=0)

    bf = 2000
    out = pl.pallas_call(
        _final_body,
        grid=(N // bf,),
        in_specs=[
            pl.BlockSpec((NC, bf, FT), lambda i: (0, i, 0)),
            pl.BlockSpec((NC, bf, HP), lambda i: (0, i, 0)),
            pl.BlockSpec((HP, FT), lambda i: (0, 0)),
            pl.BlockSpec((bf, FT), lambda i: (i, 0)),
            pl.BlockSpec((bf, FT), lambda i: (i, 0)),
        ],
        out_specs=pl.BlockSpec((bf, FT), lambda i: (i, 0)),
        out_shape=jax.ShapeDtypeStruct((N, FT), f32),
    )(accp, dpart, R, pdst, nft)
    return out


# speedup vs baseline: 1.0542x; 1.0542x over previous
import functools
import jax
import jax.numpy as jnp
from jax import lax
from jax.experimental import pallas as pl
from jax.experimental.pallas import tpu as pltpu
from jax.experimental.pallas import tpu_sc as plsc

N = 10000
E = 320000
FT = 128
H = 8
DH = FT // H
HP = 16

NC = 2
NS = 16
NW = NC * NS
EP = E // NW
C = 40
NCH = EP // C
SR = 640
SRL = N - (NS - 1) * SR

_mesh = plsc.VectorSubcoreMesh(core_axis_name="c", subcore_axis_name="s")


def _split_rows(s, do):
    @pl.when(s < NS - 1)
    def _():
        do(s * SR, SR)

    @pl.when(s == NS - 1)
    def _():
        do((NS - 1) * SR, SRL)


def _node_prep_body(nft_ref, w1t_ref, w3t_ref, wa_ref, a2s_ref, a2d_ref,
                    psrc_ref, pdst_ref, nsw_ref, ndw_ref):
    x = nft_ref[...]
    psrc = jnp.dot(x, w1t_ref[...], preferred_element_type=jnp.float32)
    pdst = jnp.dot(x, w3t_ref[...], preferred_element_type=jnp.float32)
    psrc_ref[...] = psrc
    pdst_ref[...] = pdst
    nsw_ref[...] = (jnp.dot(x, wa_ref[...], preferred_element_type=jnp.float32)
                    + jnp.dot(psrc, a2s_ref[...], preferred_element_type=jnp.float32))
    ndw_ref[...] = jnp.dot(pdst, a2d_ref[...], preferred_element_type=jnp.float32)


def _edge_prep_body(eft_ref, w2t_ref, b_ref, a2_ref, pe_ref, e2_ref):
    pe = jnp.dot(eft_ref[...], w2t_ref[...],
                 preferred_element_type=jnp.float32) + b_ref[...]
    pe_ref[...] = pe
    e2_ref[...] = jnp.dot(pe, a2_ref[...], preferred_element_type=jnp.float32)


def _final_body(accp_ref, dpart_ref, r_ref, pdst_ref, nft_ref, out_ref):
    d = jnp.dot(dpart_ref[0] + dpart_ref[1], r_ref[...],
                preferred_element_type=jnp.float32)
    inv = jnp.where(d > 0.0, 1.0 / d, 0.0)
    pd = jnp.where(d > 0.0, pdst_ref[...], 0.0)
    out_ref[...] = jnp.maximum(
        (accp_ref[0] + accp_ref[1]) * inv + pd + nft_ref[...], 0.0)


def _sc_edges(src_h, dst_h, psrc_h, nsw_h, ndw_h, pe_h, e2_h,
              p_h, accp_h,
              idx_s, idx_d, gs, gn, gd, pev, e2v, wbuf, acc,
              sem_g0, sem_g1, sem_s0, sem_s1):
    sems_g = (sem_g0, sem_g1)
    sems_s = (sem_s0, sem_s1)
    c = lax.axis_index("c")
    s = lax.axis_index("s")
    wid = s * NC + c
    base_w = wid * EP

    wb0 = wbuf.at[0]

    def zrow(i, carry):
        def zcol(h, hc):
            wb0[i, pl.ds(h * HP, HP)] = jnp.zeros((HP,), jnp.float32)
            return hc
        lax.fori_loop(0, FT // HP, zcol, 0)
        return carry
    lax.fori_loop(0, C, zrow, 0)

    def stage(off, size):
        def zcopy(j, carry):
            pltpu.sync_copy(wb0, acc.at[pl.ds(off + j * C, C), :])
            return carry
        lax.fori_loop(0, size // C, zcopy, 0)
    _split_rows(s, stage)
    plsc.subcore_barrier()

    def issue_gathers(k, b):
        off = base_w + k * C
        pltpu.sync_copy(src_h.at[pl.ds(off, C)], idx_s.at[b])
        pltpu.sync_copy(dst_h.at[pl.ds(off, C)], idx_d.at[b])
        pltpu.async_copy(psrc_h.at[idx_s.at[b]], gs.at[b], sems_g[b])
        pltpu.async_copy(nsw_h.at[idx_s.at[b]], gn.at[b], sems_g[b])
        pltpu.async_copy(ndw_h.at[idx_d.at[b]], gd.at[b], sems_g[b])
        pltpu.async_copy(pe_h.at[pl.ds(off, C), :], pev.at[b], sems_g[b])
        pltpu.async_copy(e2_h.at[pl.ds(off, C), :], e2v.at[b], sems_g[b])

    def wait_gathers(k, b):
        off = base_w + k * C
        pltpu.make_async_copy(psrc_h.at[idx_s.at[b]], gs.at[b], sems_g[b]).wait()
        pltpu.make_async_copy(nsw_h.at[idx_s.at[b]], gn.at[b], sems_g[b]).wait()
        pltpu.make_async_copy(ndw_h.at[idx_d.at[b]], gd.at[b], sems_g[b]).wait()
        pltpu.make_async_copy(pe_h.at[pl.ds(off, C), :], pev.at[b], sems_g[b]).wait()
        pltpu.make_async_copy(e2_h.at[pl.ds(off, C), :], e2v.at[b], sems_g[b]).wait()

    def issue_scatters(k, b):
        off = base_w + k * C
        pltpu.sync_copy(e2v.at[b], p_h.at[pl.ds(off, C), :])
        pltpu.sync_copy(wbuf.at[b], acc.at[idx_d.at[b]], add=True)

    def compute(b):
        gsb, gnb, gdb = gs.at[b], gn.at[b], gd.at[b]
        pevb, e2b, wbb = pev.at[b], e2v.at[b], wbuf.at[b]

        def erow(i, icarry):
            a = gnb[i, :] + gdb[i, :] + e2b[i, :]
            a = jnp.where(a >= 0.0, a, a * 0.01)
            p = jnp.exp(a)
            e2b[i, :] = p
            for h in range(H):
                w = (gsb[i, pl.ds(h * DH, DH)] + pevb[i, pl.ds(h * DH, DH)]) * p[h]
                wbb[i, pl.ds(h * DH, DH)] = w
            return icarry
        lax.fori_loop(0, C, erow, 0)

    issue_gathers(0, 0)

    def pair(kk, carry):
        for b in range(2):
            k = kk * 2 + b
            o = 1 - b

            @pl.when(k + 1 < NCH)
            def _():
                issue_gathers(k + 1, o)

            wait_gathers(k, b)
            compute(b)
            issue_scatters(k, b)
        return carry
    lax.fori_loop(0, NCH // 2, pair, 0)

    plsc.subcore_barrier()

    def writeback(off, size):
        pltpu.sync_copy(acc.at[pl.ds(off, size), :],
                        accp_h.at[c, pl.ds(off, size), :])
    _split_rows(s, writeback)


C2 = 80
NCH2 = EP // C2


def _sc_denom(dst_h, p_h,
              dpart_h,
              idx_d, pv, dsh, sem):
    c = lax.axis_index("c")
    s = lax.axis_index("s")
    wid = s * NC + c
    base_w = wid * EP

    def zrow(i, carry):
        pv[i, :] = jnp.zeros((HP,), jnp.float32)
        return carry
    lax.fori_loop(0, C2, zrow, 0)

    def stage(off, size):
        def zcopy(j, carry):
            pltpu.sync_copy(pv, dsh.at[pl.ds(off + j * C2, C2), :])
            return carry
        lax.fori_loop(0, size // C2, zcopy, 0)
    _split_rows(s, stage)
    plsc.subcore_barrier()

    def chunk(k, carry):
        off = base_w + k * C2
        pltpu.sync_copy(dst_h.at[pl.ds(off, C2)], idx_d)
        pltpu.sync_copy(p_h.at[pl.ds(off, C2), :], pv)
        pltpu.sync_copy(pv, dsh.at[idx_d], add=True)
        return carry
    lax.fori_loop(0, NCH2, chunk, 0)

    plsc.subcore_barrier()

    @pl.when(s == 0)
    def _():
        pltpu.sync_copy(dsh, dpart_h.at[c])


@jax.jit
def kernel(nft, eft, edge_index, W_pe, b_pe, W_a1, attn2):
    f32 = jnp.float32
    src = edge_index[0]
    dst = edge_index[1]
    W1T = W_pe[:, :FT].T
    W2T = W_pe[:, FT:2 * FT].T
    W3T = W_pe[:, 2 * FT:].T
    a2sq = attn2[0]
    A2 = (a2sq[:, :, None] * jnp.eye(H, dtype=f32)[:, None, :]).reshape(FT, H)
    zpad = jnp.zeros((FT, FT - H), f32)
    A2w = jnp.concatenate([A2, zpad], axis=1)
    Wa1w = jnp.concatenate([W_a1.T, zpad], axis=1)
    A2p = A2w[:, :HP]
    b2 = b_pe[None, :]

    bn = 2000
    psrc, pdst, nsw, ndw = pl.pallas_call(
        _node_prep_body,
        grid=(N // bn,),
        in_specs=[
            pl.BlockSpec((bn, FT), lambda i: (i, 0)),
            pl.BlockSpec((FT, FT), lambda i: (0, 0)),
            pl.BlockSpec((FT, FT), lambda i: (0, 0)),
            pl.BlockSpec((FT, FT), lambda i: (0, 0)),
            pl.BlockSpec((FT, FT), lambda i: (0, 0)),
            pl.BlockSpec((FT, FT), lambda i: (0, 0)),
        ],
        out_specs=[
            pl.BlockSpec((bn, FT), lambda i: (i, 0)),
            pl.BlockSpec((bn, FT), lambda i: (i, 0)),
            pl.BlockSpec((bn, FT), lambda i: (i, 0)),
            pl.BlockSpec((bn, FT), lambda i: (i, 0)),
        ],
        out_shape=[
            jax.ShapeDtypeStruct((N, FT), f32),
            jax.ShapeDtypeStruct((N, FT), f32),
            jax.ShapeDtypeStruct((N, FT), f32),
            jax.ShapeDtypeStruct((N, FT), f32),
        ],
    )(nft, W1T, W3T, Wa1w, A2w, A2w)

    be = 2000
    pe, e216 = pl.pallas_call(
        _edge_prep_body,
        grid=(E // be,),
        in_specs=[
            pl.BlockSpec((be, FT), lambda i: (i, 0)),
            pl.BlockSpec((FT, FT), lambda i: (0, 0)),
            pl.BlockSpec((1, FT), lambda i: (0, 0)),
            pl.BlockSpec((FT, HP), lambda i: (0, 0)),
        ],
        out_specs=[
            pl.BlockSpec((be, FT), lambda i: (i, 0)),
            pl.BlockSpec((be, HP), lambda i: (i, 0)),
        ],
        out_shape=[
            jax.ShapeDtypeStruct((E, FT), f32),
            jax.ShapeDtypeStruct((E, HP), f32),
        ],
    )(eft, W2T, b2, A2p)

    p_exp, accp = pl.kernel(
        _sc_edges,
        out_type=[
            jax.ShapeDtypeStruct((E, HP), f32),
            jax.ShapeDtypeStruct((NC, N, FT), f32),
        ],
        mesh=_mesh,
        compiler_params=pltpu.CompilerParams(use_tc_tiling_on_sc=False),
        scratch_types=[
            pltpu.VMEM((2, C), jnp.int32),
            pltpu.VMEM((2, C), jnp.int32),
            pltpu.VMEM((2, C, FT), f32),
            pltpu.VMEM((2, C, HP), f32),
            pltpu.VMEM((2, C, HP), f32),
            pltpu.VMEM((2, C, FT), f32),
            pltpu.VMEM((2, C, HP), f32),
            pltpu.VMEM((2, C, FT), f32),
            pltpu.VMEM_SHARED((N, FT), f32),
            pltpu.SemaphoreType.DMA,
            pltpu.SemaphoreType.DMA,
            pltpu.SemaphoreType.DMA,
            pltpu.SemaphoreType.DMA,
        ],
    )(src, dst, psrc, nsw[:, :HP], ndw[:, :HP], pe, e216)

    dpart = pl.kernel(
        _sc_denom,
        out_type=jax.ShapeDtypeStruct((NC, N, HP), f32),
        mesh=_mesh,
        compiler_params=pltpu.CompilerParams(use_tc_tiling_on_sc=False),
        scratch_types=[
            pltpu.VMEM((C2,), jnp.int32),
            pltpu.VMEM((C2, HP), f32),
            pltpu.VMEM_SHARED((N, HP), f32),
            pltpu.SemaphoreType.DMA,
        ],
    )(dst, p_exp)

    R = jnp.concatenate(
        [jnp.kron(jnp.eye(H, dtype=f32), jnp.ones((1, DH), f32)),
         jnp.zeros((HP - H, FT), f32)], axis=0)

    bf = 2000
    out = pl.pallas_call(
        _final_body,
        grid=(N // bf,),
        in_specs=[
            pl.BlockSpec((NC, bf, FT), lambda i: (0, i, 0)),
            pl.BlockSpec((NC, bf, HP), lambda i: (0, i, 0)),
            pl.BlockSpec((HP, FT), lambda i: (0, 0)),
            pl.BlockSpec((bf, FT), lambda i: (i, 0)),
            pl.BlockSpec((bf, FT), lambda i: (i, 0)),
        ],
        out_specs=pl.BlockSpec((bf, FT), lambda i: (i, 0)),
        out_shape=jax.ShapeDtypeStruct((N, FT), f32),
    )(accp, dpart, R, pdst, nft)
    return out
